# baseline (speedup 1.0000x reference)
import jax
import jax.numpy as jnp
from jax.experimental import pallas as pl
from jax.experimental.pallas import tpu as pltpu


def _w(out):
    out[...] = jnp.full(out.shape, 7, jnp.int32)


def kernel(hidden, mask, time_delta, Wq, bq, Wk, bk, Wv, bv, Wd, bd, ln_w, ln_b, emb):
    B = hidden.shape[0]
    V = emb.shape[0]
    VB = 4096
    nvb = pl.cdiv(V, VB)
    return pl.pallas_call(
        _w,
        grid=(nvb,),
        out_specs=pl.BlockSpec((B, VB), lambda j: (0, j)),
        out_shape=jax.ShapeDtypeStruct((B, V), jnp.int32),
    )()


# speedup vs baseline: 1.0023x; 1.0023x over previous
import functools

import jax
import jax.numpy as jnp
import numpy as np
from jax.experimental import pallas as pl
from jax.experimental.pallas import tpu as pltpu

_BF = jnp.bfloat16


def _attn(h, h0, m0, wq, bq, wkv2, bkv2, wd4, bd, lnw, lnb, s4, et,
          *, num_l, inv_sqrt_dh):
    D = wq.shape[0]
    q0 = (jnp.dot(h0.astype(_BF), wq[...].astype(_BF),
                  preferred_element_type=jnp.float32) + bq[...]) * inv_sqrt_dh
    q4 = jnp.concatenate([q0, q0, q0, q0], axis=1).astype(_BF)
    am0 = (m0[...] > 0).astype(jnp.float32)
    s4m = s4[...].astype(_BF)
    etm = et[...]
    eA, eB = etm[:4], etm[4:]
    both = eA + eB
    m = None
    d = None
    acc = None
    for p in range(num_l // 2):
        chunk = h[:, pl.dslice(p * 2 * D, 2 * D)]
        kv2 = jnp.dot(chunk, wkv2[...],
                      preferred_element_type=jnp.float32) + bkv2[...]
        att2 = jnp.dot(q4 * kv2.astype(_BF), s4m,
                       preferred_element_type=jnp.float32)
        for half in range(2):
            l = 2 * p + half
            att = att2[:, half * 4:(half + 1) * 4]
            ee = (eA, eB)[half]
            if l == 0:
                att = att + (-1e4) * (1.0 - am0)
                m = att
                d = jnp.ones_like(att)
                acc = jnp.dot(d, ee) * kv2
            else:
                att = att - 1e4
                m_new = jnp.maximum(m, att)
                alpha = jnp.exp(m - m_new)
                e = jnp.exp(att - m_new)
                m = m_new
                d = d * alpha + e
                acc = acc * jnp.dot(alpha, both) + jnp.dot(e, ee) * kv2

    denom = jnp.dot(d, both)
    denom = denom + (denom == 0.0)
    ctx = acc / denom
    hs = jnp.dot(ctx.astype(_BF), wd4[...].astype(_BF),
                 preferred_element_type=jnp.float32) + bd[...]
    x = hs + h0
    mu = jnp.mean(x, axis=1, keepdims=True)
    xc = x - mu
    var = jnp.mean(xc * xc, axis=1, keepdims=True)
    xn = xc / jnp.sqrt(var + 1e-12)
    return (lnw[...] * xn + lnb[...]).astype(_BF)


def _body(h, h0_ref, m0, wq, bq, wkv2, bkv2, wd4, bd, lnw, lnb, s4, et, emb,
          out, sel_s, *, num_l, inv_sqrt_dh):
    j = pl.program_id(0)

    @pl.when(j == 0)
    def _():
        sel_s[...] = _attn(h, h0_ref[...], m0, wq, bq, wkv2, bkv2, wd4, bd,
                           lnw, lnb, s4, et,
                           num_l=num_l, inv_sqrt_dh=inv_sqrt_dh)

    out[...] = jax.lax.dot_general(
        sel_s[...], emb[...].astype(_BF), (((1,), (1,)), ((), ())),
        preferred_element_type=jnp.float32).astype(_BF)


def kernel(hidden, mask, time_delta, Wq, bq, Wk, bk, Wv, bv, Wd, bd, ln_w, ln_b, emb):
    B, L, D = hidden.shape
    V = emb.shape[0]
    H = 4
    DH = D // H

    hflat = hidden.astype(_BF).reshape(B, L * D)
    h0 = hidden[:, 0, :]
    m0 = mask[:, 0].reshape(B, 1)
    wkv = jnp.concatenate([Wk, Wv], axis=1)
    z = jnp.zeros_like(wkv)
    wkv2 = jnp.concatenate(
        [jnp.concatenate([wkv, z], axis=1),
         jnp.concatenate([z, wkv], axis=1)], axis=0)
    bkv = jnp.concatenate([bk, bv])
    bkv2 = jnp.concatenate([bkv, bkv]).reshape(1, 4 * D)
    eye = jnp.repeat(jnp.eye(H, dtype=jnp.float32), DH, axis=0)
    zy = jnp.zeros_like(eye)
    s4 = jnp.concatenate(
        [jnp.concatenate([eye, zy], axis=1),
         jnp.concatenate([zy, zy], axis=1),
         jnp.concatenate([zy, eye], axis=1),
         jnp.concatenate([zy, zy], axis=1)], axis=0)
    etA = jnp.concatenate([zy, eye, zy, zy], axis=0).T
    etB = jnp.concatenate([zy, zy, zy, eye], axis=0).T
    et = jnp.concatenate([etA, etB], axis=0)
    wd4 = jnp.concatenate(
        [jnp.zeros_like(Wd), Wd, jnp.zeros_like(Wd), Wd], axis=0)
    b2 = lambda v: v.reshape(1, D)

    const = lambda j: (0, 0)
    VB = 4096
    nvb = pl.cdiv(V, VB)
    scores = pl.pallas_call(
        functools.partial(_body, num_l=L, inv_sqrt_dh=1.0 / np.sqrt(DH)),
        grid=(nvb,),
        in_specs=[
            pl.BlockSpec((B, L * D), const),
            pl.BlockSpec((B, D), const),
            pl.BlockSpec((B, 1), const),
            pl.BlockSpec((D, D), const),
            pl.BlockSpec((1, D), const),
            pl.BlockSpec((2 * D, 4 * D), const),
            pl.BlockSpec((1, 4 * D), const),
            pl.BlockSpec((4 * D, D), const),
            pl.BlockSpec((1, D), const),
            pl.BlockSpec((1, D), const),
            pl.BlockSpec((1, D), const),
            pl.BlockSpec((4 * D, 2 * H), const),
            pl.BlockSpec((2 * H, 4 * D), const),
            pl.BlockSpec((VB, D), lambda j: (j, 0)),
        ],
        out_specs=pl.BlockSpec((B, VB), lambda j: (0, j)),
        out_shape=jax.ShapeDtypeStruct((B, V), _BF),
        scratch_shapes=[pltpu.VMEM((B, D), _BF)],
    )(hflat, h0, m0, Wq, b2(bq), wkv2, bkv2, wd4, b2(bd), b2(ln_w), b2(ln_b),
      s4, et, emb)
    return scores.astype(jnp.float32)
